# baseline (speedup 1.0000x reference)
import functools

import jax
import jax.numpy as jnp
from jax import lax
from jax.experimental import pallas as pl
from jax.experimental.pallas import tpu as pltpu
from jax.experimental.pallas import tpu_sc as plsc

N = 10000
E = 160000
D = 768
R = 8

NC = 2
NS = 16
CW = 16
NCHUNK = D // CW
CPC = NCHUNK // NC
EPT = E // NS
BB = 128
GRP = 5
NB = 80
EPAD = NB * BB
NGRP = NB // GRP
NK = N * R
NKPAD = NK + BB
RPT = NKPAD // NS
ORT = NK // NS
ZR = 512


def _sc_agg_body(x_hbm, src_hbm, key_hbm, zeros_hbm, out_hbm,
                 acc, src_v, key_v, bank0, bank1, zbuf,
                 gsem0, gsem1, ssem0, ssem1):
    co = lax.axis_index("c")
    s = lax.axis_index("s")
    pltpu.sync_copy(src_hbm.at[s], src_v)
    pltpu.sync_copy(key_hbm.at[s], key_v)
    pltpu.sync_copy(zeros_hbm, zbuf)
    banks = (bank0, bank1)
    gsems = (gsem0, gsem1)
    ssems = (ssem0, ssem1)

    def chunk_body(cl, carry):
        c = co * CPC + cl
        for z in range(9):
            pltpu.sync_copy(zbuf, acc.at[pl.ds(s * RPT + z * ZR, ZR)])
        pltpu.sync_copy(zbuf.at[pl.ds(0, RPT - 9 * ZR)],
                        acc.at[pl.ds(s * RPT + 9 * ZR, RPT - 9 * ZR)])
        plsc.subcore_barrier()

        table = x_hbm.at[c]

        def fire_gathers(g, p):
            hs = []
            for j in range(GRP):
                b = g * GRP + j
                dst = banks[p].at[pl.ds(j * BB, BB)]
                hs.append(pltpu.async_copy(table.at[src_v.at[b]], dst,
                                           gsems[p]))
            return hs

        gh = [fire_gathers(0, 0), None]
        sh = [[], []]
        for g in range(NGRP):
            p = g % 2
            if g + 1 < NGRP:
                for h in sh[1 - p]:
                    h.wait()
                sh[1 - p] = []
                gh[1 - p] = fire_gathers(g + 1, 1 - p)
            for h in gh[p]:
                h.wait()
            for j in range(GRP):
                b = g * GRP + j
                srcb = banks[p].at[pl.ds(j * BB, BB)]
                sh[p].append(pltpu.async_copy(srcb, acc.at[key_v.at[b]],
                                              ssems[p], add=True))
        for p in (0, 1):
            for h in sh[p]:
                h.wait()
        plsc.subcore_barrier()
        pltpu.sync_copy(acc.at[pl.ds(s * ORT, ORT)],
                        out_hbm.at[c // 2].at[pl.ds(s * ORT, ORT),
                                              pl.ds((c % 2) * CW, CW)])
        plsc.subcore_barrier()
        return carry

    lax.fori_loop(0, CPC, chunk_body, 0)


@functools.cache
def _get_sc_agg():
    mesh = plsc.VectorSubcoreMesh(core_axis_name="c", subcore_axis_name="s",
                                  num_cores=NC, num_subcores=NS)
    return pl.kernel(
        _sc_agg_body,
        out_type=jax.ShapeDtypeStruct((NCHUNK // 2, NK, 2 * CW), jnp.float32),
        mesh=mesh,
        scratch_types=[
            pltpu.VMEM_SHARED((NKPAD, CW), jnp.float32),
            pltpu.VMEM((NB, BB), jnp.int32),
            pltpu.VMEM((NB, BB), jnp.int32),
            pltpu.VMEM((GRP * BB, CW), jnp.float32),
            pltpu.VMEM((GRP * BB, CW), jnp.float32),
            pltpu.VMEM((ZR, CW), jnp.float32),
            pltpu.SemaphoreType.DMA,
            pltpu.SemaphoreType.DMA,
            pltpu.SemaphoreType.DMA,
            pltpu.SemaphoreType.DMA,
        ],
        compiler_params=pltpu.CompilerParams(use_tc_tiling_on_sc=False),
    )


def _sc_cnt_body(key_hbm, ones_hbm, zeros_hbm, out_hbm,
                 acc, key_v, bank, zbuf, ssem):
    co = lax.axis_index("c")
    s = lax.axis_index("s")

    @pl.when(co == 0)
    def _():
        pltpu.sync_copy(key_hbm.at[s], key_v)
        pltpu.sync_copy(ones_hbm, bank)
        pltpu.sync_copy(zeros_hbm, zbuf)
        for z in range(9):
            pltpu.sync_copy(zbuf, acc.at[pl.ds(s * RPT + z * ZR, ZR)])
        pltpu.sync_copy(zbuf.at[pl.ds(0, RPT - 9 * ZR)],
                        acc.at[pl.ds(s * RPT + 9 * ZR, RPT - 9 * ZR)])
        plsc.subcore_barrier()
        for g in range(NGRP):
            hs = []
            for j in range(GRP):
                b = g * GRP + j
                hs.append(pltpu.async_copy(bank.at[pl.ds(j * BB, BB)],
                                           acc.at[key_v.at[b]], ssem,
                                           add=True))
            for h in hs:
                h.wait()
        plsc.subcore_barrier()
        pltpu.sync_copy(acc.at[pl.ds(s * ORT, ORT)],
                        out_hbm.at[pl.ds(s * ORT, ORT)])


@functools.cache
def _get_sc_cnt():
    mesh = plsc.VectorSubcoreMesh(core_axis_name="c", subcore_axis_name="s",
                                  num_cores=NC, num_subcores=NS)
    return pl.kernel(
        _sc_cnt_body,
        out_type=jax.ShapeDtypeStruct((NK, CW), jnp.float32),
        mesh=mesh,
        scratch_types=[
            pltpu.VMEM_SHARED((NKPAD, CW), jnp.float32),
            pltpu.VMEM((NB, BB), jnp.int32),
            pltpu.VMEM((GRP * BB, CW), jnp.float32),
            pltpu.VMEM((ZR, CW), jnp.float32),
            pltpu.SemaphoreType.DMA,
        ],
        compiler_params=pltpu.CompilerParams(use_tc_tiling_on_sc=False),
    )


BN = 1000
NBLK = N // BN


KB = 2 * BB
NKB = NCHUNK // 2


def _mm_body(a_ref, cnt_ref, x_ref, w_ref, r_ref, b_ref, o_ref, acc_ref,
             *, relu):
    k = pl.program_id(1)

    @pl.when(k == 0)
    def _():
        acc_ref[...] = jnp.zeros_like(acc_ref)

    a = a_ref[...].reshape(BN, KB)
    cnt = cnt_ref[...].reshape(BN, KB)
    a = a * (1.0 / jnp.maximum(cnt, 1.0))
    w = w_ref[...].reshape(KB, D)
    acc_ref[...] += jnp.dot(a, w, preferred_element_type=jnp.float32)

    @pl.when(k < D // KB)
    def _():
        acc_ref[...] += jnp.dot(x_ref[...], r_ref[0],
                                preferred_element_type=jnp.float32)

    @pl.when(k == NKB - 1)
    def _():
        o = acc_ref[...] + b_ref[0]
        o_ref[...] = jnp.maximum(o, 0.0) if relu else o


def _make_mm(relu):
    nroot = D // KB
    return pl.pallas_call(
        functools.partial(_mm_body, relu=relu),
        grid=(NBLK, NKB),
        in_specs=[
            pl.BlockSpec((BN * KB,), lambda n, k: (k * NBLK + n,)),
            pl.BlockSpec((BN * KB,), lambda n, k: (n,)),
            pl.BlockSpec((BN, KB), lambda n, k: (n, jnp.minimum(k, nroot - 1))),
            pl.BlockSpec((R, 1, 2, CW, D), lambda n, k: (0, k, 0, 0, 0)),
            pl.BlockSpec((1, KB, D), lambda n, k: (jnp.minimum(k, nroot - 1), 0, 0)),
            pl.BlockSpec((1, D), lambda n, k: (0, 0)),
        ],
        out_specs=pl.BlockSpec((BN, D), lambda n, k: (n, 0)),
        out_shape=jax.ShapeDtypeStruct((N, D), jnp.float32),
        scratch_shapes=[pltpu.VMEM((BN, D), jnp.float32)],
        compiler_params=pltpu.CompilerParams(
            dimension_semantics=("parallel", "arbitrary")),
    )


_mm_relu = _make_mm(True)
_mm_lin = _make_mm(False)


def kernel(entity, edge_index, edge_type, W, root, bias):
    src = edge_index[0].astype(jnp.int32)
    dst = edge_index[1].astype(jnp.int32)
    key = dst * R + edge_type.astype(jnp.int32)

    pad = jnp.arange(EPAD - EPT, dtype=jnp.int32)
    srcT = src.reshape(NS, EPT)
    src_pad = jnp.broadcast_to((pad * 1009) % N, (NS, EPAD - EPT))
    srcTp = jnp.concatenate([srcT, src_pad], axis=1).reshape(NS, NB, BB)
    keyT = key.reshape(NS, EPT)
    key_pad = jnp.broadcast_to(NK + pad % BB, (NS, EPAD - EPT))
    keyTp = jnp.concatenate([keyT, key_pad], axis=1).reshape(NS, NB, BB)

    zeros = jnp.zeros((ZR, CW), jnp.float32)
    ones = jnp.ones((GRP * BB, CW), jnp.float32)

    cnt16 = _get_sc_cnt()(keyTp, ones, zeros)
    cntv = cnt16.reshape(N, R, CW)[:, :, 0]
    cnt = jnp.broadcast_to(cntv.reshape(N, R, 1, 1),
                           (N, R, 2, CW)).reshape(-1)

    nroot = D // KB
    x = entity
    for l in range(3):
        xt3 = x.reshape(N, NCHUNK, CW).transpose(1, 0, 2)
        a3 = _get_sc_agg()(xt3, srcTp, keyTp, zeros).reshape(-1)
        w4 = W[l].reshape(R, NCHUNK // 2, 2, CW, D)
        root6 = root[l].reshape(nroot, KB, D)
        b2 = bias[l].reshape(1, D)
        mm = _mm_relu if l < 2 else _mm_lin
        x = mm(a3, cnt, x, w4, root6, b2)
    return x


# speedup vs baseline: 1.3267x; 1.3267x over previous
import functools

import jax
import jax.numpy as jnp
from jax import lax
from jax.experimental import pallas as pl
from jax.experimental.pallas import tpu as pltpu
from jax.experimental.pallas import tpu_sc as plsc

N = 10000
E = 160000
D = 768
R = 8

NC = 2
NS = 16
CW = 16
NCHUNK = D // CW
CPC = NCHUNK // NC
EPT = E // NS
BB = 128
GRP = 5
NB = 80
EPAD = NB * BB
NGRP = NB // GRP
NK = N * R
NKPAD = NK + BB
RPT = NKPAD // NS
ORT = NK // NS
ZR = 512


def _sc_agg_body(x_hbm, src_hbm, key_hbm, zeros_hbm, out_hbm,
                 acc, src_v, key_v, bank0, bank1, zbuf,
                 gsem0, gsem1, ssem0, ssem1):
    co = lax.axis_index("c")
    s = lax.axis_index("s")
    pltpu.sync_copy(src_hbm.at[s], src_v)
    pltpu.sync_copy(key_hbm.at[s], key_v)
    pltpu.sync_copy(zeros_hbm, zbuf)
    banks = (bank0, bank1)
    gsems = (gsem0, gsem1)
    ssems = (ssem0, ssem1)

    def chunk_body(cl, carry):
        c = co * CPC + cl
        for z in range(9):
            pltpu.sync_copy(zbuf, acc.at[pl.ds(s * RPT + z * ZR, ZR)])
        pltpu.sync_copy(zbuf.at[pl.ds(0, RPT - 9 * ZR)],
                        acc.at[pl.ds(s * RPT + 9 * ZR, RPT - 9 * ZR)])
        plsc.subcore_barrier()

        table = x_hbm.at[c]

        def fire_gathers(g, p):
            hs = []
            for j in range(GRP):
                b = g * GRP + j
                dst = banks[p].at[pl.ds(j * BB, BB)]
                hs.append(pltpu.async_copy(table.at[src_v.at[b]], dst,
                                           gsems[p]))
            return hs

        gh = [fire_gathers(0, 0), None]
        sh = [[], []]
        for g in range(NGRP):
            p = g % 2
            if g + 1 < NGRP:
                for h in sh[1 - p]:
                    h.wait()
                sh[1 - p] = []
                gh[1 - p] = fire_gathers(g + 1, 1 - p)
            for h in gh[p]:
                h.wait()
            for j in range(GRP):
                b = g * GRP + j
                srcb = banks[p].at[pl.ds(j * BB, BB)]
                sh[p].append(pltpu.async_copy(srcb, acc.at[key_v.at[b]],
                                              ssems[p], add=True))
        for p in (0, 1):
            for h in sh[p]:
                h.wait()
        plsc.subcore_barrier()
        pltpu.sync_copy(acc.at[pl.ds(s * ORT, ORT)],
                        out_hbm.at[c].at[pl.ds(s * ORT, ORT)])
        plsc.subcore_barrier()
        return carry

    lax.fori_loop(0, CPC, chunk_body, 0)


@functools.cache
def _get_sc_agg():
    mesh = plsc.VectorSubcoreMesh(core_axis_name="c", subcore_axis_name="s",
                                  num_cores=NC, num_subcores=NS)
    return pl.kernel(
        _sc_agg_body,
        out_type=jax.ShapeDtypeStruct((NCHUNK, NK, CW), jnp.float32),
        mesh=mesh,
        scratch_types=[
            pltpu.VMEM_SHARED((NKPAD, CW), jnp.float32),
            pltpu.VMEM((NB, BB), jnp.int32),
            pltpu.VMEM((NB, BB), jnp.int32),
            pltpu.VMEM((GRP * BB, CW), jnp.float32),
            pltpu.VMEM((GRP * BB, CW), jnp.float32),
            pltpu.VMEM((ZR, CW), jnp.float32),
            pltpu.SemaphoreType.DMA,
            pltpu.SemaphoreType.DMA,
            pltpu.SemaphoreType.DMA,
            pltpu.SemaphoreType.DMA,
        ],
        compiler_params=pltpu.CompilerParams(use_tc_tiling_on_sc=False),
    )


def _sc_cnt_body(key_hbm, ones_hbm, zeros_hbm, out_hbm,
                 acc, key_v, bank, zbuf, ssem):
    co = lax.axis_index("c")
    s = lax.axis_index("s")

    @pl.when(co == 0)
    def _():
        pltpu.sync_copy(key_hbm.at[s], key_v)
        pltpu.sync_copy(ones_hbm, bank)
        pltpu.sync_copy(zeros_hbm, zbuf)
        for z in range(9):
            pltpu.sync_copy(zbuf, acc.at[pl.ds(s * RPT + z * ZR, ZR)])
        pltpu.sync_copy(zbuf.at[pl.ds(0, RPT - 9 * ZR)],
                        acc.at[pl.ds(s * RPT + 9 * ZR, RPT - 9 * ZR)])
        plsc.subcore_barrier()
        for g in range(NGRP):
            hs = []
            for j in range(GRP):
                b = g * GRP + j
                hs.append(pltpu.async_copy(bank.at[pl.ds(j * BB, BB)],
                                           acc.at[key_v.at[b]], ssem,
                                           add=True))
            for h in hs:
                h.wait()
        plsc.subcore_barrier()
        pltpu.sync_copy(acc.at[pl.ds(s * ORT, ORT)],
                        out_hbm.at[pl.ds(s * ORT, ORT)])


@functools.cache
def _get_sc_cnt():
    mesh = plsc.VectorSubcoreMesh(core_axis_name="c", subcore_axis_name="s",
                                  num_cores=NC, num_subcores=NS)
    return pl.kernel(
        _sc_cnt_body,
        out_type=jax.ShapeDtypeStruct((NK, CW), jnp.float32),
        mesh=mesh,
        scratch_types=[
            pltpu.VMEM_SHARED((NKPAD, CW), jnp.float32),
            pltpu.VMEM((NB, BB), jnp.int32),
            pltpu.VMEM((GRP * BB, CW), jnp.float32),
            pltpu.VMEM((ZR, CW), jnp.float32),
            pltpu.SemaphoreType.DMA,
        ],
        compiler_params=pltpu.CompilerParams(use_tc_tiling_on_sc=False),
    )


BN = 2000
NBLK = N // BN


def _mm_body(a_ref, cnt_ref, x_ref, w_ref, r_ref, b_ref, o_ref, acc_ref,
             *, relu):
    k = pl.program_id(1)

    @pl.when(k == 0)
    def _():
        acc_ref[...] = jnp.zeros_like(acc_ref)

    a = a_ref[...].reshape(BN, BB)
    cnt = cnt_ref[...].reshape(BN, BB)
    a = a * (1.0 / jnp.maximum(cnt, 1.0))
    w = w_ref[...].reshape(BB, D)
    acc_ref[...] += jnp.dot(a, w, preferred_element_type=jnp.float32)

    @pl.when(k < D // BB)
    def _():
        acc_ref[...] += jnp.dot(x_ref[...], r_ref[0],
                                preferred_element_type=jnp.float32)

    @pl.when(k == NCHUNK - 1)
    def _():
        o = acc_ref[...] + b_ref[0]
        o_ref[...] = jnp.maximum(o, 0.0) if relu else o


def _make_mm(relu):
    nroot = D // BB
    return pl.pallas_call(
        functools.partial(_mm_body, relu=relu),
        grid=(NBLK, NCHUNK),
        in_specs=[
            pl.BlockSpec((BN * BB,), lambda n, k: (k * NBLK + n,)),
            pl.BlockSpec((BN * BB,), lambda n, k: (n,)),
            pl.BlockSpec((BN, BB), lambda n, k: (n, jnp.minimum(k, nroot - 1))),
            pl.BlockSpec((R, 1, CW, D), lambda n, k: (0, k, 0, 0)),
            pl.BlockSpec((1, BB, D), lambda n, k: (jnp.minimum(k, nroot - 1), 0, 0)),
            pl.BlockSpec((1, D), lambda n, k: (0, 0)),
        ],
        out_specs=pl.BlockSpec((BN, D), lambda n, k: (n, 0)),
        out_shape=jax.ShapeDtypeStruct((N, D), jnp.float32),
        scratch_shapes=[pltpu.VMEM((BN, D), jnp.float32)],
        compiler_params=pltpu.CompilerParams(
            dimension_semantics=("parallel", "arbitrary")),
    )


_mm_relu = _make_mm(True)
_mm_lin = _make_mm(False)


def kernel(entity, edge_index, edge_type, W, root, bias):
    src = edge_index[0].astype(jnp.int32)
    dst = edge_index[1].astype(jnp.int32)
    key = dst * R + edge_type.astype(jnp.int32)

    pad = jnp.arange(EPAD - EPT, dtype=jnp.int32)
    srcT = src.reshape(NS, EPT)
    src_pad = jnp.broadcast_to((pad * 1009) % N, (NS, EPAD - EPT))
    srcTp = jnp.concatenate([srcT, src_pad], axis=1).reshape(NS, NB, BB)
    keyT = key.reshape(NS, EPT)
    key_pad = jnp.broadcast_to(NK + pad % BB, (NS, EPAD - EPT))
    keyTp = jnp.concatenate([keyT, key_pad], axis=1).reshape(NS, NB, BB)

    zeros = jnp.zeros((ZR, CW), jnp.float32)
    ones = jnp.ones((GRP * BB, CW), jnp.float32)

    cnt = _get_sc_cnt()(keyTp, ones, zeros).reshape(-1)

    nroot = D // BB
    x = entity
    for l in range(3):
        xt3 = x.reshape(N, NCHUNK, CW).transpose(1, 0, 2)
        a3 = _get_sc_agg()(xt3, srcTp, keyTp, zeros).reshape(-1)
        w4 = W[l].reshape(R, NCHUNK, CW, D)
        root6 = root[l].reshape(nroot, BB, D)
        b2 = bias[l].reshape(1, D)
        mm = _mm_relu if l < 2 else _mm_lin
        x = mm(a3, cnt, x, w4, root6, b2)
    return x
